# baseline (speedup 1.0000x reference)
import functools

import jax
import jax.numpy as jnp
import numpy as np
from jax import lax
from jax.experimental import pallas as pl
from jax.experimental.pallas import tpu as pltpu
from jax.experimental.pallas import tpu_sc as plsc

_SLOW_FRAMES = 8
_NUM_CORES = 2
_NUM_SUBCORES = 16


def kernel(frames):
    B, T, C, H, W = frames.shape
    S = _SLOW_FRAMES
    F = C * H * W
    idx = np.linspace(0, T - 1, S).astype(np.int32)

    flat = frames.reshape(B * T, F)
    n_rows = B * S
    nw = _NUM_CORES * _NUM_SUBCORES
    rows_per_w = -(-n_rows // nw)

    mesh = plsc.VectorSubcoreMesh(core_axis_name="c", subcore_axis_name="s")

    @functools.partial(
        pl.kernel,
        mesh=mesh,
        out_type=jax.ShapeDtypeStruct((n_rows, F), jnp.float32),
        scratch_types=[pltpu.SemaphoreType.DMA],
    )
    def gather_k(in_hbm, out_hbm, sem):
        wid = lax.axis_index("s") * _NUM_CORES + lax.axis_index("c")
        copies = []
        for r in range(rows_per_w):
            i = wid * rows_per_w + r
            b = i // S
            j = i - b * S
            t_src = jnp.int32(int(idx[S - 1]))
            for jj in range(S - 1):
                t_src = jnp.where(j == jj, jnp.int32(int(idx[jj])), t_src)
            src = b * T + t_src
            cp = pltpu.make_async_copy(in_hbm.at[src], out_hbm.at[i], sem)
            cp.start()
            copies.append(cp)
        for cp in copies:
            cp.wait()

    slow = gather_k(flat).reshape(B, S, C, H, W)
    return (slow, frames)


# speedup vs baseline: 8.0409x; 8.0409x over previous
import functools

import jax
import jax.numpy as jnp
import numpy as np
from jax.experimental import pallas as pl

_SLOW_FRAMES = 8


def kernel(frames):
    B, T, C, H, W = frames.shape
    S = _SLOW_FRAMES
    idx = [int(v) for v in np.linspace(0, T - 1, S).astype(np.int32)]

    def body(in_ref, slow_ref, fast_ref):
        data = in_ref[...]
        fast_ref[...] = data
        t = pl.program_id(1)
        sel = functools.reduce(
            lambda a, c: a | (t == c), idx[1:], t == idx[0]
        )

        @pl.when(sel)
        def _():
            slow_ref[...] = data

    def slow_map(b, t):
        j = functools.reduce(
            lambda a, c: a + (t >= c).astype(jnp.int32), idx[1:],
            (t >= idx[0]).astype(jnp.int32),
        ) - 1
        return (b, j, 0, 0, 0)

    slow, fast = pl.pallas_call(
        body,
        grid=(B, T),
        in_specs=[
            pl.BlockSpec((1, 1, C, H, W), lambda b, t: (b, t, 0, 0, 0)),
        ],
        out_specs=[
            pl.BlockSpec((1, 1, C, H, W), slow_map),
            pl.BlockSpec((1, 1, C, H, W), lambda b, t: (b, t, 0, 0, 0)),
        ],
        out_shape=[
            jax.ShapeDtypeStruct((B, S, C, H, W), frames.dtype),
            jax.ShapeDtypeStruct((B, T, C, H, W), frames.dtype),
        ],
    )(frames)
    return (slow, fast)


# speedup vs baseline: 13.6332x; 1.6955x over previous
import jax
import jax.numpy as jnp
import numpy as np
from jax.experimental import pallas as pl

_SLOW_FRAMES = 8


def _pick_tb(T, S, idx):
    for tb in (8, 4, 2, 1):
        if T % tb or S % (T // tb):
            continue
        per = S // (T // tb)
        counts = [sum(1 for v in idx if blk * tb <= v < (blk + 1) * tb)
                  for blk in range(T // tb)]
        if all(c == per for c in counts):
            return tb, per
    return 1, None


def _select_const(x, table):
    out = jnp.int32(table[-1])
    for i in range(len(table) - 1):
        out = jnp.where(x == i, jnp.int32(table[i]), out)
    return out


def kernel(frames):
    B, T, C, H, W = frames.shape
    S = _SLOW_FRAMES
    idx = [int(v) for v in np.linspace(0, T - 1, S).astype(np.int32)]
    Tb, per = _pick_tb(T, S, idx)
    nblk = T // Tb
    offs = [[idx[tb * per + k] - tb * Tb for tb in range(nblk)]
            for k in range(per)]

    def body(in_ref, slow_ref, fast_ref):
        data = in_ref[...]
        fast_ref[...] = data
        tb = pl.program_id(1)
        for k in range(per):
            off = _select_const(tb, offs[k])
            slow_ref[0, k] = in_ref[0, off]

    slow, fast = pl.pallas_call(
        body,
        grid=(B, nblk),
        in_specs=[
            pl.BlockSpec((1, Tb, C, H, W), lambda b, t: (b, t, 0, 0, 0)),
        ],
        out_specs=[
            pl.BlockSpec((1, per, C, H, W), lambda b, t: (b, t, 0, 0, 0)),
            pl.BlockSpec((1, Tb, C, H, W), lambda b, t: (b, t, 0, 0, 0)),
        ],
        out_shape=[
            jax.ShapeDtypeStruct((B, S, C, H, W), frames.dtype),
            jax.ShapeDtypeStruct((B, T, C, H, W), frames.dtype),
        ],
    )(frames)
    return (slow, fast)


# speedup vs baseline: 13.8520x; 1.0161x over previous
import jax
import jax.numpy as jnp
import numpy as np
from jax.experimental import pallas as pl

_SLOW_FRAMES = 8


def _pick_tb(T, S, idx):
    for tb in (16, 8, 4, 2, 1):
        if T % tb or S % (T // tb):
            continue
        per = S // (T // tb)
        counts = [sum(1 for v in idx if blk * tb <= v < (blk + 1) * tb)
                  for blk in range(T // tb)]
        if all(c == per for c in counts):
            return tb, per
    return 1, None


def _select_const(x, table):
    out = jnp.int32(table[-1])
    for i in range(len(table) - 1):
        out = jnp.where(x == i, jnp.int32(table[i]), out)
    return out


def kernel(frames):
    B, T, C, H, W = frames.shape
    S = _SLOW_FRAMES
    idx = [int(v) for v in np.linspace(0, T - 1, S).astype(np.int32)]
    Tb, per = _pick_tb(T, S, idx)
    nblk = T // Tb
    offs = [[idx[tb * per + k] - tb * Tb for tb in range(nblk)]
            for k in range(per)]

    def body(in_ref, slow_ref, fast_ref):
        data = in_ref[...]
        fast_ref[...] = data
        tb = pl.program_id(1)
        for k in range(per):
            off = _select_const(tb, offs[k])
            slow_ref[0, k] = in_ref[0, off]

    slow, fast = pl.pallas_call(
        body,
        grid=(B, nblk),
        in_specs=[
            pl.BlockSpec((1, Tb, C, H, W), lambda b, t: (b, t, 0, 0, 0)),
        ],
        out_specs=[
            pl.BlockSpec((1, per, C, H, W), lambda b, t: (b, t, 0, 0, 0)),
            pl.BlockSpec((1, Tb, C, H, W), lambda b, t: (b, t, 0, 0, 0)),
        ],
        out_shape=[
            jax.ShapeDtypeStruct((B, S, C, H, W), frames.dtype),
            jax.ShapeDtypeStruct((B, T, C, H, W), frames.dtype),
        ],
    )(frames)
    return (slow, fast)
